# baseline (speedup 1.0000x reference)
import functools

import jax
import jax.numpy as jnp
import numpy as np
from jax import lax
from jax.experimental import pallas as pl
from jax.experimental.pallas import tpu as pltpu
from jax.experimental.pallas import tpu_sc as plsc

_N_TP = 3
_P = 15
_D = 512
_HID = 512
_NA = 32
_T = 128
_A = 16
_N = _T * _A
_TYPE = np.array([0] * 8 + [1] * 4 + [2] * 4, dtype=np.int32)
_BLK = 128
_POOLS = ((0, 1024, 0, 12), (1024, 512, 12, 8), (1536, 512, 20, 8))
_EBLOCKS = 28
_CBLOCKS = _N // _BLK
_GRID = _EBLOCKS

_NC = 2
_NS = 16
_NW = _NC * _NS


def _routing_metadata(hete_pick):
    tp = jnp.asarray(_TYPE)[None, :]
    ph = (hete_pick.astype(jnp.int32) * _N_TP + tp).reshape(-1)
    onehot = (ph[:, None] == jnp.arange(_P, dtype=jnp.int32)[None, :]).astype(jnp.int32)
    ranks = jnp.cumsum(onehot, axis=0)
    counts = ranks[-1]
    rank = jnp.sum(ranks * onehot, axis=1) - 1
    c53 = counts.reshape(5, 3)
    nb53 = (c53 + _BLK - 1) // _BLK
    cum53 = jnp.cumsum(nb53, axis=0)
    base_blocks = jnp.asarray(np.array([p[2] for p in _POOLS], np.int32))
    poff53 = (cum53 - nb53 + base_blocks[None, :]) * _BLK
    poff = poff53.reshape(_P)
    tok_pos = jnp.sum(onehot * poff[None, :], axis=1) + rank
    eids = []
    for t, (_, _, b0, nb) in enumerate(_POOLS):
        lb = jnp.arange(nb, dtype=jnp.int32)
        g = jnp.sum((lb[:, None] >= cum53[None, :, t]).astype(jnp.int32), axis=1)
        eids.append(3 * jnp.minimum(g, 4) + t)
    blk_eid = jnp.concatenate(eids)
    return tok_pos.astype(jnp.int32), blk_eid.astype(jnp.int32)


def _pool_perm_cols(a2):
    return jnp.concatenate([a2[:, :8].reshape(-1, *a2.shape[2:]),
                            a2[:, 8:12].reshape(-1, *a2.shape[2:]),
                            a2[:, 12:16].reshape(-1, *a2.shape[2:])], axis=0)


@functools.lru_cache(maxsize=None)
def _make_sc_gather(n_out, d, n_chunks):
    assert n_out % (8 * _NW) == 0 and d % 16 == 0
    b_per_w = n_out // _NW
    assert b_per_w % n_chunks == 0 and (b_per_w // n_chunks) % 8 == 0
    chunk = b_per_w // n_chunks
    mesh = plsc.VectorSubcoreMesh(core_axis_name="c", subcore_axis_name="s",
                                  num_cores=_NC, num_subcores=_NS)

    @functools.partial(
        pl.kernel, mesh=mesh,
        out_type=jax.ShapeDtypeStruct((n_out, d), jnp.float32),
        scratch_types=[
            pltpu.VMEM((b_per_w,), jnp.int32),
            pltpu.VMEM((b_per_w, d), jnp.float32),
            pltpu.SemaphoreType.DMA,
        ],
    )
    def gather(table_hbm, idx_hbm, out_hbm, idx_v, rows_v, sem):
        wid = lax.axis_index("s") * _NC + lax.axis_index("c")
        base = wid * b_per_w
        pltpu.sync_copy(idx_hbm.at[pl.ds(base, b_per_w)], idx_v)
        copies = [
            pltpu.async_copy(table_hbm.at[idx_v.at[pl.ds(c * chunk, chunk)]],
                             rows_v.at[pl.ds(c * chunk, chunk)], sem)
            for c in range(n_chunks)
        ]
        for cp in copies:
            cp.wait()
        pltpu.sync_copy(rows_v, out_hbm.at[pl.ds(base, b_per_w)])

    return gather


_LAN = 128


def _sc_unsort(table, idx):
    return _make_sc_gather(_N, _LAN, 8)(table, idx)


def _mlp_body(eid_ref, tok_pos_ref, x_ref, xhi_ref, xlo_ref,
              w1_ref, w2_ref, wa_ref, b1_ref, b2_ref, ba_ref,
              vw1_ref, vb1_ref, vw2_ref, vb2_ref, vw3_ref, vb3_ref,
              oe_ref, ov_ref):
    i = pl.program_id(0)

    for off, k, b0, nb in _POOLS:
        @pl.when(jnp.logical_and(i >= b0, i < b0 + nb))
        def _expert(off=off, k=k):
            eid = eid_ref[i]
            sel = tok_pos_ref[:, off:off + k] - i * _BLK
            row = jax.lax.broadcasted_iota(jnp.int32, (_BLK, k), 0)
            onehot = (row == sel).astype(jnp.bfloat16)
            xb = (jnp.dot(onehot, xhi_ref[off:off + k, :], preferred_element_type=jnp.float32)
                  + jnp.dot(onehot, xlo_ref[off:off + k, :], preferred_element_type=jnp.float32))
            b1 = b1_ref[pl.ds(eid, 1), :]
            b2 = b2_ref[pl.ds(eid, 1), :]
            ba = ba_ref[pl.ds(eid, 1), :]
            h = jnp.maximum(jnp.dot(xb, w1_ref[0], preferred_element_type=jnp.float32) + b1, 0.0)
            h = jnp.maximum(jnp.dot(h, w2_ref[0], preferred_element_type=jnp.float32) + b2, 0.0)
            res = jnp.dot(h, wa_ref[0], preferred_element_type=jnp.float32) + ba
            oe_ref[...] = jnp.concatenate(
                [res, jnp.zeros((_BLK, _LAN - _NA), jnp.float32)], axis=1)

    @pl.when(i < _CBLOCKS)
    def _critic():
        xb = x_ref[pl.ds(i * _BLK, _BLK), :]
        h = jnp.maximum(jnp.dot(xb, vw1_ref[...], preferred_element_type=jnp.float32) + vb1_ref[...], 0.0)
        h = jnp.maximum(jnp.dot(h, vw2_ref[...], preferred_element_type=jnp.float32) + vb2_ref[...], 0.0)
        val = jnp.dot(h, vw3_ref[...], preferred_element_type=jnp.float32) + vb3_ref[...]
        ov_ref[...] = val


def _mlp(tok_pos_pool, blk_eid, x_pool, W1, b1, W2, b2, Wa, ba,
         Vw1, Vb1, Vw2, Vb2, Vw3, Vb3):
    ew = lambda i, eid: (eid[jnp.minimum(i, _EBLOCKS - 1)], 0, 0)
    full = lambda i, eid: (0, 0)
    grid_spec = pltpu.PrefetchScalarGridSpec(
        num_scalar_prefetch=1,
        grid=(_GRID,),
        in_specs=[
            pl.BlockSpec((1, _N), full),
            pl.BlockSpec((_N, _D), full),
            pl.BlockSpec((_N, _D), full),
            pl.BlockSpec((_N, _D), full),
            pl.BlockSpec((1, _D, _HID), ew),
            pl.BlockSpec((1, _HID, _HID), ew),
            pl.BlockSpec((1, _HID, _NA), ew),
            pl.BlockSpec((_P, _HID), full),
            pl.BlockSpec((_P, _HID), full),
            pl.BlockSpec((_P, _NA), full),
            pl.BlockSpec((_D, _HID), full),
            pl.BlockSpec((1, _HID), full),
            pl.BlockSpec((_HID, _HID), full),
            pl.BlockSpec((1, _HID), full),
            pl.BlockSpec((_HID, 1), full),
            pl.BlockSpec((1, 1), full),
        ],
        out_specs=[
            pl.BlockSpec((_BLK, _LAN), lambda i, eid: (i, 0)),
            pl.BlockSpec((_BLK, 1), lambda i, eid: (jnp.minimum(i, _CBLOCKS), 0)),
        ],
    )
    x_hi = x_pool.astype(jnp.bfloat16)
    x_lo = (x_pool - x_hi.astype(jnp.float32)).astype(jnp.bfloat16)
    return pl.pallas_call(
        _mlp_body, grid_spec=grid_spec,
        out_shape=[
            jax.ShapeDtypeStruct((_EBLOCKS * _BLK, _LAN), jnp.float32),
            jax.ShapeDtypeStruct(((_CBLOCKS + 1) * _BLK, 1), jnp.float32),
        ],
    )(blk_eid, tok_pos_pool.reshape(1, _N), x_pool, x_hi, x_lo,
      W1, W2, Wa, b1, b2, ba,
      Vw1, Vb1.reshape(1, _HID), Vw2, Vb2.reshape(1, _HID), Vw3, Vb3.reshape(1, 1))


def kernel(obs, hete_pick, W1, b1, W2, b2, Wa, ba, Vw1, Vb1, Vw2, Vb2, Vw3, Vb3):
    tok_pos, blk_eid = _routing_metadata(hete_pick)
    x_pool = _pool_perm_cols(obs)
    tok_pos_pool = _pool_perm_cols(tok_pos.reshape(_T, _A, 1))[:, 0]
    logits_sorted, val_pool = _mlp(tok_pos_pool, blk_eid, x_pool,
                                   W1, b1, W2, b2, Wa, ba,
                                   Vw1, Vb1, Vw2, Vb2, Vw3, Vb3)
    logits = _sc_unsort(logits_sorted, tok_pos)[:, :_NA]
    v = val_pool[:_N]
    val = jnp.concatenate([v[:1024].reshape(_T, 8), v[1024:1536].reshape(_T, 4),
                           v[1536:].reshape(_T, 4)], axis=1).reshape(_N, 1)
    return jnp.concatenate([logits, val], axis=-1).reshape(_T, _A, _NA + 1)


# speedup vs baseline: 1.0962x; 1.0962x over previous
import functools

import jax
import jax.numpy as jnp
import numpy as np
from jax import lax
from jax.experimental import pallas as pl
from jax.experimental.pallas import tpu as pltpu
from jax.experimental.pallas import tpu_sc as plsc

_N_TP = 3
_P = 15
_D = 512
_HID = 512
_NA = 32
_T = 128
_A = 16
_N = _T * _A
_TYPE = np.array([0] * 8 + [1] * 4 + [2] * 4, dtype=np.int32)
_BLK = 128
_POOLS = ((0, 1024, 0, 12), (1024, 512, 12, 8), (1536, 512, 20, 8))
_EBLOCKS = 28
_CBLOCKS = _N // _BLK
_GRID = _EBLOCKS

_NC = 2
_NS = 16
_NW = _NC * _NS


def _routing_metadata(hete_pick):
    tp = jnp.asarray(_TYPE)[None, :]
    ph = (hete_pick.astype(jnp.int32) * _N_TP + tp).reshape(-1)
    onehot = (ph[:, None] == jnp.arange(_P, dtype=jnp.int32)[None, :]).astype(jnp.int32)
    ranks = jnp.cumsum(onehot, axis=0)
    counts = ranks[-1]
    rank = jnp.sum(ranks * onehot, axis=1) - 1
    c53 = counts.reshape(5, 3)
    nb53 = (c53 + _BLK - 1) // _BLK
    cum53 = jnp.cumsum(nb53, axis=0)
    base_blocks = jnp.asarray(np.array([p[2] for p in _POOLS], np.int32))
    poff53 = (cum53 - nb53 + base_blocks[None, :]) * _BLK
    poff = poff53.reshape(_P)
    tok_pos = jnp.sum(onehot * poff[None, :], axis=1) + rank
    eids = []
    for t, (_, _, b0, nb) in enumerate(_POOLS):
        lb = jnp.arange(nb, dtype=jnp.int32)
        g = jnp.sum((lb[:, None] >= cum53[None, :, t]).astype(jnp.int32), axis=1)
        eids.append(3 * jnp.minimum(g, 4) + t)
    blk_eid = jnp.concatenate(eids)
    return tok_pos.astype(jnp.int32), blk_eid.astype(jnp.int32)


def _pool_perm_cols(a2):
    return jnp.concatenate([a2[:, :8].reshape(-1, *a2.shape[2:]),
                            a2[:, 8:12].reshape(-1, *a2.shape[2:]),
                            a2[:, 12:16].reshape(-1, *a2.shape[2:])], axis=0)


@functools.lru_cache(maxsize=None)
def _make_sc_gather(n_out, d, n_chunks):
    assert n_out % (8 * _NW) == 0 and d % 16 == 0
    b_per_w = n_out // _NW
    assert b_per_w % n_chunks == 0 and (b_per_w // n_chunks) % 8 == 0
    chunk = b_per_w // n_chunks
    mesh = plsc.VectorSubcoreMesh(core_axis_name="c", subcore_axis_name="s",
                                  num_cores=_NC, num_subcores=_NS)

    @functools.partial(
        pl.kernel, mesh=mesh,
        out_type=jax.ShapeDtypeStruct((n_out, d), jnp.float32),
        scratch_types=[
            pltpu.VMEM((b_per_w,), jnp.int32),
            pltpu.VMEM((b_per_w, d), jnp.float32),
            pltpu.SemaphoreType.DMA,
        ],
    )
    def gather(table_hbm, idx_hbm, out_hbm, idx_v, rows_v, sem):
        wid = lax.axis_index("s") * _NC + lax.axis_index("c")
        base = wid * b_per_w
        pltpu.sync_copy(idx_hbm.at[pl.ds(base, b_per_w)], idx_v)
        copies = [
            pltpu.async_copy(table_hbm.at[idx_v.at[pl.ds(c * chunk, chunk)]],
                             rows_v.at[pl.ds(c * chunk, chunk)], sem)
            for c in range(n_chunks)
        ]
        for cp in copies:
            cp.wait()
        pltpu.sync_copy(rows_v, out_hbm.at[pl.ds(base, b_per_w)])

    return gather


_LAN = 128


def _sc_unsort(table, idx):
    return _make_sc_gather(_N, _LAN, 8)(table, idx)


def _mlp_body(eid_ref, tok_pos_ref, x_ref,
              w1_ref, w2_ref, wa_ref, b1_ref, b2_ref, ba_ref,
              vw1_ref, vb1_ref, vw2_ref, vb2_ref, vw3_ref, vb3_ref,
              oe_ref, ov_ref):
    i = pl.program_id(0)

    for off, k, b0, nb in _POOLS:
        @pl.when(jnp.logical_and(i >= b0, i < b0 + nb))
        def _expert(off=off, k=k):
            eid = eid_ref[i]
            sel = tok_pos_ref[:, off:off + k] - i * _BLK
            row = jax.lax.broadcasted_iota(jnp.int32, (_BLK, k), 0)
            onehot = (row == sel).astype(jnp.float32)
            xb = jnp.dot(onehot, x_ref[off:off + k, :], preferred_element_type=jnp.float32)
            b1 = b1_ref[pl.ds(eid, 1), :]
            b2 = b2_ref[pl.ds(eid, 1), :]
            ba = ba_ref[pl.ds(eid, 1), :]
            h = jnp.maximum(jnp.dot(xb, w1_ref[0], preferred_element_type=jnp.float32) + b1, 0.0)
            h = jnp.maximum(jnp.dot(h, w2_ref[0], preferred_element_type=jnp.float32) + b2, 0.0)
            res = jnp.dot(h, wa_ref[0], preferred_element_type=jnp.float32) + ba
            oe_ref[...] = jnp.concatenate(
                [res, jnp.zeros((_BLK, _LAN - _NA), jnp.float32)], axis=1)

    @pl.when(i < _CBLOCKS)
    def _critic():
        xb = x_ref[pl.ds(i * _BLK, _BLK), :]
        h = jnp.maximum(jnp.dot(xb, vw1_ref[...], preferred_element_type=jnp.float32) + vb1_ref[...], 0.0)
        h = jnp.maximum(jnp.dot(h, vw2_ref[...], preferred_element_type=jnp.float32) + vb2_ref[...], 0.0)
        val = jnp.dot(h, vw3_ref[...], preferred_element_type=jnp.float32) + vb3_ref[...]
        ov_ref[...] = val


def _mlp(tok_pos_pool, blk_eid, x_pool, W1, b1, W2, b2, Wa, ba,
         Vw1, Vb1, Vw2, Vb2, Vw3, Vb3):
    ew = lambda i, eid: (eid[jnp.minimum(i, _EBLOCKS - 1)], 0, 0)
    full = lambda i, eid: (0, 0)
    grid_spec = pltpu.PrefetchScalarGridSpec(
        num_scalar_prefetch=1,
        grid=(_GRID,),
        in_specs=[
            pl.BlockSpec((1, _N), full),
            pl.BlockSpec((_N, _D), full),
            pl.BlockSpec((1, _D, _HID), ew),
            pl.BlockSpec((1, _HID, _HID), ew),
            pl.BlockSpec((1, _HID, _NA), ew),
            pl.BlockSpec((_P, _HID), full),
            pl.BlockSpec((_P, _HID), full),
            pl.BlockSpec((_P, _NA), full),
            pl.BlockSpec((_D, _HID), full),
            pl.BlockSpec((1, _HID), full),
            pl.BlockSpec((_HID, _HID), full),
            pl.BlockSpec((1, _HID), full),
            pl.BlockSpec((_HID, 1), full),
            pl.BlockSpec((1, 1), full),
        ],
        out_specs=[
            pl.BlockSpec((_BLK, _LAN), lambda i, eid: (i, 0)),
            pl.BlockSpec((_BLK, 1), lambda i, eid: (jnp.minimum(i, _CBLOCKS), 0)),
        ],
    )
    return pl.pallas_call(
        _mlp_body, grid_spec=grid_spec,
        out_shape=[
            jax.ShapeDtypeStruct((_EBLOCKS * _BLK, _LAN), jnp.float32),
            jax.ShapeDtypeStruct(((_CBLOCKS + 1) * _BLK, 1), jnp.float32),
        ],
    )(blk_eid, tok_pos_pool.reshape(1, _N), x_pool,
      W1, W2, Wa, b1, b2, ba,
      Vw1, Vb1.reshape(1, _HID), Vw2, Vb2.reshape(1, _HID), Vw3, Vb3.reshape(1, 1))


def kernel(obs, hete_pick, W1, b1, W2, b2, Wa, ba, Vw1, Vb1, Vw2, Vb2, Vw3, Vb3):
    tok_pos, blk_eid = _routing_metadata(hete_pick)
    x_pool = _pool_perm_cols(obs)
    tok_pos_pool = _pool_perm_cols(tok_pos.reshape(_T, _A, 1))[:, 0]
    logits_sorted, val_pool = _mlp(tok_pos_pool, blk_eid, x_pool,
                                   W1, b1, W2, b2, Wa, ba,
                                   Vw1, Vb1, Vw2, Vb2, Vw3, Vb3)
    logits = _sc_unsort(logits_sorted, tok_pos)[:, :_NA]
    v = val_pool[:_N]
    val = jnp.concatenate([v[:1024].reshape(_T, 8), v[1024:1536].reshape(_T, 4),
                           v[1536:].reshape(_T, 4)], axis=1).reshape(_N, 1)
    return jnp.concatenate([logits, val], axis=-1).reshape(_T, _A, _NA + 1)
